# baseline (speedup 1.0000x reference)
import functools

import jax
import jax.numpy as jnp
from jax import lax
from jax.experimental import pallas as pl
from jax.experimental.pallas import tpu as pltpu
from jax.experimental.pallas import tpu_sc as plsc

HIDDEN = 128
TABLE_ROWS = 24
B_TOTAL = 512 * 4 * 16 * 16
CHUNK_R = 256
IDXC = 128
STREAM_R = 224
TEC_R = CHUNK_R - STREAM_R
LANES = 16


def _make_gather():
    info = plsc.get_sparse_core_info()
    nc, ns = info.num_cores, info.num_subcores
    nw = nc * ns
    b_per_w = B_TOTAL // nw
    n_chunks = b_per_w // CHUNK_R
    mesh = plsc.VectorSubcoreMesh(core_axis_name="c", subcore_axis_name="s")

    @functools.partial(
        pl.kernel,
        mesh=mesh,
        out_type=jax.ShapeDtypeStruct((B_TOTAL, HIDDEN), jnp.float32),
        scratch_types=[
            pltpu.VMEM_SHARED((TABLE_ROWS, HIDDEN), jnp.float32),
            pltpu.VMEM((TABLE_ROWS, HIDDEN), jnp.float32),
            pltpu.VMEM((b_per_w,), jnp.int32),
            pltpu.VMEM((CHUNK_R, HIDDEN), jnp.float32),
            pltpu.VMEM((CHUNK_R, HIDDEN), jnp.float32),
            pltpu.SemaphoreType.DMA,
            pltpu.SemaphoreType.DMA,
        ],
        compiler_params=pltpu.CompilerParams(needs_layout_passes=False),
    )
    def gather_kernel(x_hbm, pe_hbm, out_hbm, table_v, table_t, idx_v, rows_a,
                      rows_b, sem_g, sem_o):
        bufs = (rows_a, rows_b)
        wid = lax.axis_index("s") * nc + lax.axis_index("c")
        base = wid * b_per_w
        @pl.when(lax.axis_index("s") == 0)
        def _():
            pltpu.sync_copy(pe_hbm, table_v)

        pltpu.sync_copy(pe_hbm, table_t)
        pltpu.sync_copy(x_hbm.at[pl.ds(base, b_per_w)], idx_v)
        plsc.subcore_barrier()

        def g_copies(s, slot):
            cps = []
            k = 0
            while k < STREAM_R:
                n = min(IDXC, STREAM_R - k)
                cps.append(pltpu.make_async_copy(
                    table_v.at[idx_v.at[pl.ds(s * CHUNK_R + k, n)]],
                    bufs[slot].at[pl.ds(k, n)],
                    sem_g))
                k += n
            return cps

        def o_copy(s, slot):
            return pltpu.make_async_copy(
                bufs[slot],
                out_hbm.at[pl.ds(base + s * CHUNK_R, CHUNK_R)],
                sem_o)

        def expand(s, slot):
            cps = g_copies(s, slot)
            for cp in cps:
                cp.start()
            buf = bufs[slot]

            @plsc.parallel_loop(0, TEC_R // LANES)
            def group(g):
                row16 = idx_v[pl.ds(s * CHUNK_R + STREAM_R + g * LANES, LANES)]
                for l in range(LANES):
                    r = row16[l]
                    dst = STREAM_R + g * LANES + l
                    for j in range(0, HIDDEN, LANES):
                        buf[dst, pl.ds(j, LANES)] = table_t[r, pl.ds(j, LANES)]

            for cp in cps:
                cp.wait()

        expand(0, 0)
        o_copy(0, 0).start()
        expand(1, 1)
        o_copy(1, 1).start()

        def body(j, carry):
            s0 = 2 + 2 * j
            o_copy(s0 - 2, 0).wait()
            expand(s0, 0)
            o_copy(s0, 0).start()
            o_copy(s0 - 1, 1).wait()
            expand(s0 + 1, 1)
            o_copy(s0 + 1, 1).start()
            return carry

        lax.fori_loop(0, (n_chunks - 2) // 2, body, 0)

        o_copy(n_chunks - 2, 0).wait()
        o_copy(n_chunks - 1, 1).wait()

    return gather_kernel


def kernel(x, pe):
    orig_shape = x.shape
    flat = x.reshape(B_TOTAL).astype(jnp.int32)
    out = _make_gather()(flat, pe)
    return out.reshape(*orig_shape, HIDDEN)


# speedup vs baseline: 1.0453x; 1.0453x over previous
import functools

import jax
import jax.numpy as jnp
from jax import lax
from jax.experimental import pallas as pl
from jax.experimental.pallas import tpu as pltpu
from jax.experimental.pallas import tpu_sc as plsc

HIDDEN = 128
TABLE_ROWS = 24
B_TOTAL = 512 * 4 * 16 * 16
CHUNK_R = 256
IDXC = 128
STREAM_R = 192
TEC_R = CHUNK_R - STREAM_R
LANES = 16


def _make_gather():
    info = plsc.get_sparse_core_info()
    nc, ns = info.num_cores, info.num_subcores
    nw = nc * ns
    b_per_w = B_TOTAL // nw
    n_chunks = b_per_w // CHUNK_R
    mesh = plsc.VectorSubcoreMesh(core_axis_name="c", subcore_axis_name="s")

    @functools.partial(
        pl.kernel,
        mesh=mesh,
        out_type=jax.ShapeDtypeStruct((B_TOTAL, HIDDEN), jnp.float32),
        scratch_types=[
            pltpu.VMEM_SHARED((TABLE_ROWS, HIDDEN), jnp.float32),
            pltpu.VMEM((TABLE_ROWS, HIDDEN), jnp.float32),
            pltpu.VMEM((b_per_w,), jnp.int32),
            pltpu.VMEM((CHUNK_R, HIDDEN), jnp.float32),
            pltpu.VMEM((CHUNK_R, HIDDEN), jnp.float32),
            pltpu.SemaphoreType.DMA,
            pltpu.SemaphoreType.DMA,
        ],
        compiler_params=pltpu.CompilerParams(needs_layout_passes=False),
    )
    def gather_kernel(x_hbm, pe_hbm, out_hbm, table_v, table_t, idx_v, rows_a,
                      rows_b, sem_g, sem_o):
        bufs = (rows_a, rows_b)
        wid = lax.axis_index("s") * nc + lax.axis_index("c")
        base = wid * b_per_w
        @pl.when(lax.axis_index("s") == 0)
        def _():
            pltpu.sync_copy(pe_hbm, table_v)

        pltpu.sync_copy(pe_hbm, table_t)
        pltpu.sync_copy(x_hbm.at[pl.ds(base, b_per_w)], idx_v)
        plsc.subcore_barrier()

        def g_copies(s, slot):
            cps = []
            k = 0
            while k < STREAM_R:
                n = min(IDXC, STREAM_R - k)
                cps.append(pltpu.make_async_copy(
                    table_v.at[idx_v.at[pl.ds(s * CHUNK_R + k, n)]],
                    bufs[slot].at[pl.ds(k, n)],
                    sem_g))
                k += n
            return cps

        def o_copy(s, slot):
            return pltpu.make_async_copy(
                bufs[slot],
                out_hbm.at[pl.ds(base + s * CHUNK_R, CHUNK_R)],
                sem_o)

        def expand(s, slot):
            cps = g_copies(s, slot)
            for cp in cps:
                cp.start()
            buf = bufs[slot]

            @plsc.parallel_loop(0, TEC_R // LANES)
            def group(g):
                row16 = idx_v[pl.ds(s * CHUNK_R + STREAM_R + g * LANES, LANES)]
                for l in range(LANES):
                    r = row16[l]
                    dst = STREAM_R + g * LANES + l
                    for j in range(0, HIDDEN, LANES):
                        buf[dst, pl.ds(j, LANES)] = table_t[r, pl.ds(j, LANES)]

            for cp in cps:
                cp.wait()

        expand(0, 0)
        o_copy(0, 0).start()
        expand(1, 1)
        o_copy(1, 1).start()

        def body(j, carry):
            s0 = 2 + 2 * j
            o_copy(s0 - 2, 0).wait()
            expand(s0, 0)
            o_copy(s0, 0).start()
            o_copy(s0 - 1, 1).wait()
            expand(s0 + 1, 1)
            o_copy(s0 + 1, 1).start()
            return carry

        lax.fori_loop(0, (n_chunks - 2) // 2, body, 0)

        o_copy(n_chunks - 2, 0).wait()
        o_copy(n_chunks - 1, 1).wait()

    return gather_kernel


def kernel(x, pe):
    orig_shape = x.shape
    flat = x.reshape(B_TOTAL).astype(jnp.int32)
    out = _make_gather()(flat, pe)
    return out.reshape(*orig_shape, HIDDEN)
